# baseline (speedup 1.0000x reference)
import functools
import math

import jax
import jax.numpy as jnp
from jax import lax
from jax.experimental import pallas as pl
from jax.experimental.pallas import tpu as pltpu
from jax.experimental.pallas import tpu_sc as plsc


def _pos_encoding(seq_len, dim):
    position = jnp.arange(0, seq_len, dtype=jnp.float32)[:, None]
    div_term = jnp.exp(
        jnp.arange(0, dim, 2, dtype=jnp.float32) * -(math.log(10000.0) / dim)
    )
    pe = jnp.zeros((seq_len, dim), dtype=jnp.float32)
    pe = pe.at[:, 0::2].set(jnp.sin(position * div_term))
    pe = pe.at[:, 1::2].set(jnp.cos(position * div_term))
    return pe


@functools.partial(jax.jit, static_argnums=(3, 4))
def _sc_embed(idx, pe, table, batch, seq):
    n_rows = batch * seq
    dim = table.shape[1]
    NC, NS = 2, 16
    NW = NC * NS
    seq_per_w = batch // NW
    s_a = 128
    s_b = seq - s_a

    mesh = plsc.VectorSubcoreMesh(core_axis_name="c", subcore_axis_name="s")

    @functools.partial(
        pl.kernel,
        mesh=mesh,
        out_type=jax.ShapeDtypeStruct((n_rows, dim), jnp.float32),
        scratch_types=[
            pltpu.VMEM((seq, dim), jnp.float32),
            pltpu.VMEM((s_a,), jnp.int32),
            pltpu.VMEM((s_b,), jnp.int32),
            pltpu.VMEM((seq, dim), jnp.float32),
            pltpu.SemaphoreType.DMA,
        ],
        compiler_params=pltpu.CompilerParams(use_tc_tiling_on_sc=False),
    )
    def body(idx_hbm, pe_hbm, table_hbm, out_hbm, pe_v, idx_a, idx_b, rows_v, sem):
        wid = lax.axis_index("s") * NC + lax.axis_index("c")
        pltpu.sync_copy(pe_hbm, pe_v)

        def chunk(t, carry):
            base = (wid * seq_per_w + t) * seq
            pltpu.sync_copy(idx_hbm.at[pl.ds(base, s_a)], idx_a)
            pltpu.sync_copy(idx_hbm.at[pl.ds(base + s_a, s_b)], idx_b)
            cp1 = pltpu.async_copy(
                table_hbm.at[idx_a], rows_v.at[pl.ds(0, s_a)], sem
            )
            cp2 = pltpu.async_copy(
                table_hbm.at[idx_b], rows_v.at[pl.ds(s_a, s_b)], sem
            )
            cp1.wait()
            cp2.wait()

            def add_row(i, c2):
                for c in range(dim // 16):
                    sl = pl.ds(c * 16, 16)
                    rows_v[i, sl] = rows_v[i, sl] + pe_v[i, sl]
                return c2

            lax.fori_loop(0, seq, add_row, 0)
            pltpu.sync_copy(rows_v, out_hbm.at[pl.ds(base, seq)])
            return carry

        lax.fori_loop(0, seq_per_w, chunk, 0)

    return body(idx, pe, table)


def kernel(x, table):
    batch, seq = x.shape
    dim = table.shape[1]
    idx = x.reshape(-1).astype(jnp.int32)
    pe = _pos_encoding(seq, dim)
    out = _sc_embed(idx, pe, table, batch, seq)
    return out.reshape(batch, seq, dim)


# speedup vs baseline: 1.2297x; 1.2297x over previous
import functools
import math

import jax
import jax.numpy as jnp
from jax import lax
from jax.experimental import pallas as pl
from jax.experimental.pallas import tpu as pltpu
from jax.experimental.pallas import tpu_sc as plsc


def _pos_encoding(seq_len, dim):
    position = jnp.arange(0, seq_len, dtype=jnp.float32)[:, None]
    div_term = jnp.exp(
        jnp.arange(0, dim, 2, dtype=jnp.float32) * -(math.log(10000.0) / dim)
    )
    pe = jnp.zeros((seq_len, dim), dtype=jnp.float32)
    pe = pe.at[:, 0::2].set(jnp.sin(position * div_term))
    pe = pe.at[:, 1::2].set(jnp.cos(position * div_term))
    return pe


@functools.partial(jax.jit, static_argnums=(3, 4))
def _sc_embed(idx, pe, table, batch, seq):
    n_rows = batch * seq
    dim = table.shape[1]
    NC, NS = 2, 16
    NW = NC * NS
    n_chunks = batch // NW
    s_a = 128
    s_b = seq - s_a

    mesh = plsc.VectorSubcoreMesh(core_axis_name="c", subcore_axis_name="s")

    @functools.partial(
        pl.kernel,
        mesh=mesh,
        out_type=jax.ShapeDtypeStruct((n_rows, dim), jnp.float32),
        scratch_types=[
            pltpu.VMEM((seq, dim), jnp.float32),
            pltpu.VMEM((seq,), jnp.int32),
            pltpu.VMEM((seq,), jnp.int32),
            pltpu.VMEM((seq, dim), jnp.float32),
            pltpu.VMEM((seq, dim), jnp.float32),
            pltpu.VMEM((seq, dim), jnp.float32),
            pltpu.VMEM((seq, dim), jnp.float32),
            pltpu.SemaphoreType.DMA,
            pltpu.SemaphoreType.DMA,
            pltpu.SemaphoreType.DMA,
            pltpu.SemaphoreType.DMA,
            pltpu.SemaphoreType.DMA,
            pltpu.SemaphoreType.DMA,
        ],
        compiler_params=pltpu.CompilerParams(use_tc_tiling_on_sc=False),
    )
    def body(idx_hbm, pe_hbm, table_hbm, out_hbm,
             pe_v, idx0, idx1, rows0, rows1, outs0, outs1,
             gs0, gs1, is0, is1, os0, os1):
        wid = lax.axis_index("s") * NC + lax.axis_index("c")
        first = wid * n_chunks
        pltpu.sync_copy(pe_hbm, pe_v)

        def issue_gather(t, idxb, rowsb, gsem):
            base = (first + t) * seq
            pltpu.async_copy(
                table_hbm.at[idxb.at[pl.ds(0, s_a)]],
                rowsb.at[pl.ds(0, s_a)], gsem)
            pltpu.async_copy(
                table_hbm.at[idxb.at[pl.ds(s_a, s_b)]],
                rowsb.at[pl.ds(s_a, s_b)], gsem)

        bufs = ((idx0, rows0, outs0, gs0, is0, os0),
                (idx1, rows1, outs1, gs1, is1, os1))

        for b in range(2):
            idxb, rowsb, _, gsem, _, _ = bufs[b]
            base = (first + b) * seq
            pltpu.sync_copy(idx_hbm.at[pl.ds(base, seq)], idxb)
            issue_gather(b, idxb, rowsb, gsem)

        def process(t, buf):
            idxb, rowsb, outb, gsem, isem, osem = buf
            pltpu.make_async_copy(
                table_hbm.at[idxb], rowsb, gsem).wait()
            @pl.when(t + 2 < n_chunks)
            def _():
                base2 = (first + t + 2) * seq
                pltpu.async_copy(idx_hbm.at[pl.ds(base2, seq)], idxb, isem)
            @pl.when(t >= 2)
            def _():
                pltpu.make_async_copy(
                    outb, out_hbm.at[pl.ds(0, seq)], osem).wait()

            @plsc.parallel_loop(0, seq, 1, unroll=8)
            def _(i):
                for c in range(dim // 16):
                    sl = pl.ds(c * 16, 16)
                    outb[i, sl] = rowsb[i, sl] + pe_v[i, sl]

            base = (first + t) * seq
            pltpu.async_copy(outb, out_hbm.at[pl.ds(base, seq)], osem)
            @pl.when(t + 2 < n_chunks)
            def _():
                pltpu.make_async_copy(
                    idx_hbm.at[pl.ds(0, seq)], idxb, isem).wait()
                issue_gather(t + 2, idxb, rowsb, gsem)

        def step(g, carry):
            process(2 * g, bufs[0])
            process(2 * g + 1, bufs[1])
            return carry

        lax.fori_loop(0, n_chunks // 2, step, 0)

        for b in range(2):
            _, _, outb, _, _, osem = bufs[b]
            pltpu.make_async_copy(outb, out_hbm.at[pl.ds(0, seq)], osem).wait()

    return body(idx, pe, table)


def kernel(x, table):
    batch, seq = x.shape
    dim = table.shape[1]
    idx = x.reshape(-1).astype(jnp.int32)
    pe = _pos_encoding(seq, dim)
    out = _sc_embed(idx, pe, table, batch, seq)
    return out.reshape(batch, seq, dim)
